# baseline (speedup 1.0000x reference)
import functools

import jax
import jax.numpy as jnp
from jax import lax
from jax.experimental import pallas as pl
from jax.experimental.pallas import tpu as pltpu
from jax.experimental.pallas import tpu_sc as plsc

IN_F = 2048
HID = 8192
OUT_F = 2048
N_TOK = 4096
K_SEL = 2048

NC = 2
NS = 16
NW = NC * NS
ROWS_PER_W = K_SEL // NW
CHUNK = 16
NCHUNK = ROWS_PER_W // CHUNK


@functools.cache
def _get_sc_gather():
    mesh = plsc.VectorSubcoreMesh(core_axis_name="c", subcore_axis_name="s",
                                  num_cores=NC, num_subcores=NS)

    @functools.partial(
        pl.kernel,
        out_type=(
            jax.ShapeDtypeStruct((K_SEL, IN_F), jnp.float32),
            jax.ShapeDtypeStruct((K_SEL,), jnp.float32),
            jax.ShapeDtypeStruct((K_SEL, OUT_F), jnp.float32),
        ),
        mesh=mesh,
        compiler_params=pltpu.CompilerParams(needs_layout_passes=False),
        scratch_types=[
            pltpu.VMEM((ROWS_PER_W,), jnp.int32),
            pltpu.VMEM((CHUNK, IN_F), jnp.float32),
            pltpu.VMEM((CHUNK, OUT_F), jnp.float32),
            pltpu.VMEM((HID,), jnp.float32),
            pltpu.VMEM((ROWS_PER_W,), jnp.float32),
            pltpu.SemaphoreType.DMA,
            pltpu.SemaphoreType.DMA,
            pltpu.SemaphoreType.DMA,
            pltpu.SemaphoreType.DMA,
        ],
    )
    def _sc_gather(fc1_w_hbm, fc1_b_hbm, fc2_w_hbm, idx_hbm,
                   w1_out, b1_out, w2_out,
                   idx_v, bufa, bufb, bias_v, bsel_v,
                   gsem_a, gsem_b, osem_a, osem_b, ):
        wid = lax.axis_index("s") * NC + lax.axis_index("c")
        base = wid * ROWS_PER_W
        pltpu.sync_copy(idx_hbm.at[pl.ds(base, ROWS_PER_W)], idx_v)

        pltpu.sync_copy(fc1_b_hbm, bias_v)
        for c in range(ROWS_PER_W // 16):
            idxs = idx_v[pl.ds(c * 16, 16)]
            bsel_v[pl.ds(c * 16, 16)] = plsc.load_gather(bias_v, [idxs])
        pltpu.sync_copy(bsel_v, b1_out.at[pl.ds(base, ROWS_PER_W)])

        steps = ([(fc1_w_hbm, w1_out, c) for c in range(NCHUNK)]
                 + [(fc2_w_hbm, w2_out, c) for c in range(NCHUNK)])
        bufs = (bufa, bufb)
        gsems = (gsem_a, gsem_b)
        osems = (osem_a, osem_b)
        gathers = [None, None]
        outs = [None, None]
        for t, (tbl, out, c) in enumerate(steps):
            b = t % 2
            if outs[b] is not None:
                outs[b].wait()
            idxs = idx_v[pl.ds(c * CHUNK, CHUNK)]
            gathers[b] = pltpu.async_copy(tbl.at[idxs], bufs[b], gsems[b])
            if t > 0:
                pb = (t - 1) % 2
                prev_tbl, prev_out, prev_c = steps[t - 1]
                gathers[pb].wait()
                outs[pb] = pltpu.async_copy(
                    bufs[pb], prev_out.at[pl.ds(base + prev_c * CHUNK, CHUNK)],
                    osems[pb])
        lb = (len(steps) - 1) % 2
        last_tbl, last_out, last_c = steps[-1]
        gathers[lb].wait()
        outs[lb] = pltpu.async_copy(
            bufs[lb], last_out.at[pl.ds(base + last_c * CHUNK, CHUNK)],
            osems[lb])
        outs[0].wait()
        outs[1].wait()

    return _sc_gather


BM = 512
KS = 256
NKS = K_SEL // KS
NT = N_TOK // BM


def _mlp_body(x_ref, w1_ref, b1_ref, w2_ref, b2_ref, o_ref,
              xbf, w1bf, w2bf):
    t = pl.program_id(0)
    s = pl.program_id(1)

    @pl.when(s == 0)
    def _cx():
        xbf[...] = x_ref[...].astype(jnp.bfloat16)

    @pl.when(t == 0)
    def _cw():
        w1bf[pl.ds(s * KS, KS), :] = w1_ref[...].astype(jnp.bfloat16)
        w2bf[pl.ds(s * KS, KS), :] = w2_ref[...].astype(jnp.bfloat16)

    h = lax.dot_general(xbf[...], w1bf[pl.ds(s * KS, KS), :],
                        (((1,), (1,)), ((), ())),
                        preferred_element_type=jnp.float32)
    h = jnp.maximum(h + b1_ref[...], 0.0).astype(jnp.bfloat16)
    part = lax.dot_general(h, w2bf[pl.ds(s * KS, KS), :],
                           (((1,), (0,)), ((), ())),
                           preferred_element_type=jnp.float32)

    @pl.when(s == 0)
    def _first():
        o_ref[...] = part + b2_ref[...]

    @pl.when(s != 0)
    def _acc():
        o_ref[...] += part


_mlp = pl.pallas_call(
    _mlp_body,
    grid=(NT, NKS),
    in_specs=[
        pl.BlockSpec((BM, IN_F), lambda t, s: (t, 0)),
        pl.BlockSpec((KS, IN_F),
                     lambda t, s: (jnp.where(t == 0, s, NKS - 1), 0)),
        pl.BlockSpec((1, KS), lambda t, s: (0, s)),
        pl.BlockSpec((KS, OUT_F),
                     lambda t, s: (jnp.where(t == 0, s, NKS - 1), 0)),
        pl.BlockSpec((1, OUT_F), lambda t, s: (0, 0)),
    ],
    out_specs=pl.BlockSpec((BM, OUT_F), lambda t, s: (t, 0)),
    out_shape=jax.ShapeDtypeStruct((N_TOK, OUT_F), jnp.float32),
    scratch_shapes=[
        pltpu.VMEM((BM, IN_F), jnp.bfloat16),
        pltpu.VMEM((K_SEL, IN_F), jnp.bfloat16),
        pltpu.VMEM((K_SEL, OUT_F), jnp.bfloat16),
    ],
    compiler_params=pltpu.CompilerParams(
        dimension_semantics=("arbitrary", "arbitrary"),
        vmem_limit_bytes=120 * 1024 * 1024,
    ),
)


def kernel(x, index_vec, fc1_w, fc1_b, fc2_w_t, fc2_b):
    idx = index_vec.astype(jnp.int32)
    w1_sel, b1_sel, w2_sel = _get_sc_gather()(fc1_w, fc1_b, fc2_w_t, idx)
    return _mlp(x, w1_sel, b1_sel.reshape(1, K_SEL), w2_sel,
                fc2_b.reshape(1, OUT_F))


# speedup vs baseline: 1.5892x; 1.5892x over previous
import functools

import jax
import jax.numpy as jnp
from jax import lax
from jax.experimental import pallas as pl
from jax.experimental.pallas import tpu as pltpu
from jax.experimental.pallas import tpu_sc as plsc

IN_F = 2048
HID = 8192
OUT_F = 2048
N_TOK = 4096
K_SEL = 2048

NC = 2
NS = 16
NW = NC * NS
ROWS_PER_W = K_SEL // NW
CHUNK = 16
NCHUNK = ROWS_PER_W // CHUNK


@functools.cache
def _get_sc_gather():
    mesh = plsc.VectorSubcoreMesh(core_axis_name="c", subcore_axis_name="s",
                                  num_cores=NC, num_subcores=NS)

    @functools.partial(
        pl.kernel,
        out_type=(
            jax.ShapeDtypeStruct((K_SEL, IN_F), jnp.float32),
            jax.ShapeDtypeStruct((K_SEL,), jnp.float32),
            jax.ShapeDtypeStruct((K_SEL, OUT_F), jnp.float32),
        ),
        mesh=mesh,
        compiler_params=pltpu.CompilerParams(needs_layout_passes=False),
        scratch_types=[
            pltpu.VMEM((ROWS_PER_W,), jnp.int32),
            pltpu.VMEM((CHUNK, IN_F), jnp.float32),
            pltpu.VMEM((CHUNK, OUT_F), jnp.float32),
            pltpu.VMEM((HID,), jnp.float32),
            pltpu.VMEM((ROWS_PER_W,), jnp.float32),
            pltpu.SemaphoreType.DMA,
            pltpu.SemaphoreType.DMA,
            pltpu.SemaphoreType.DMA,
            pltpu.SemaphoreType.DMA,
        ],
    )
    def _sc_gather(fc1_w_hbm, fc1_b_hbm, fc2_w_hbm, idx_hbm,
                   w1_out, b1_out, w2_out,
                   idx_v, bufa, bufb, bias_v, bsel_v,
                   gsem_a, gsem_b, osem_a, osem_b, ):
        wid = lax.axis_index("s") * NC + lax.axis_index("c")
        base = wid * ROWS_PER_W
        pltpu.sync_copy(idx_hbm.at[pl.ds(base, ROWS_PER_W)], idx_v)

        pltpu.sync_copy(fc1_b_hbm, bias_v)
        for c in range(ROWS_PER_W // 16):
            idxs = idx_v[pl.ds(c * 16, 16)]
            bsel_v[pl.ds(c * 16, 16)] = plsc.load_gather(bias_v, [idxs])
        pltpu.sync_copy(bsel_v, b1_out.at[pl.ds(base, ROWS_PER_W)])

        steps = ([(fc1_w_hbm, w1_out, c) for c in range(NCHUNK)]
                 + [(fc2_w_hbm, w2_out, c) for c in range(NCHUNK)])
        bufs = (bufa, bufb)
        gsems = (gsem_a, gsem_b)
        osems = (osem_a, osem_b)
        gathers = [None, None]
        outs = [None, None]
        for t, (tbl, out, c) in enumerate(steps):
            b = t % 2
            if outs[b] is not None:
                outs[b].wait()
            idxs = idx_v[pl.ds(c * CHUNK, CHUNK)]
            gathers[b] = pltpu.async_copy(tbl.at[idxs], bufs[b], gsems[b])
            if t > 0:
                pb = (t - 1) % 2
                prev_tbl, prev_out, prev_c = steps[t - 1]
                gathers[pb].wait()
                outs[pb] = pltpu.async_copy(
                    bufs[pb], prev_out.at[pl.ds(base + prev_c * CHUNK, CHUNK)],
                    osems[pb])
        lb = (len(steps) - 1) % 2
        last_tbl, last_out, last_c = steps[-1]
        gathers[lb].wait()
        outs[lb] = pltpu.async_copy(
            bufs[lb], last_out.at[pl.ds(base + last_c * CHUNK, CHUNK)],
            osems[lb])
        outs[0].wait()
        outs[1].wait()

    return _sc_gather


BM = 512
STAGE = 256
NSTAGE = K_SEL // STAGE


def _cast_x_body(x_ref, o_ref):
    o_ref[...] = x_ref[...].astype(jnp.bfloat16)


_cast_x = pl.pallas_call(
    _cast_x_body,
    grid=(8,),
    in_specs=[pl.BlockSpec((N_TOK // 8, IN_F), lambda i: (i, 0))],
    out_specs=pl.BlockSpec((N_TOK // 8, IN_F), lambda i: (i, 0)),
    out_shape=jax.ShapeDtypeStruct((N_TOK, IN_F), jnp.bfloat16),
    compiler_params=pltpu.CompilerParams(dimension_semantics=("arbitrary",)),
)


def _wslice(src, s, stg, sem):
    return pltpu.make_async_copy(
        src.at[pl.ds(s * STAGE, STAGE), :], stg, sem)


def _mlp_body(x_ref, w1_any, b1_ref, w2_any, b2_ref, o_ref,
              w1bf, w2bf, stg_a, stg_b, sem_a, sem_b, sem_c, sem_d):
    i = pl.program_id(0)
    stgs = (stg_a, stg_b)
    sems1 = (sem_a, sem_b)
    sems2 = (sem_c, sem_d)

    @pl.when(i == 0)
    def _init_w1():
        for s in range(NSTAGE):
            b = s % 2
            _wslice(w1_any, s, stgs[b], sems1[b]).start()
            if s > 0:
                pb = 1 - b
                _wslice(w1_any, s - 1, stgs[pb], sems1[pb]).wait()
                w1bf[pl.ds((s - 1) * STAGE, STAGE), :] = (
                    stgs[pb][...].astype(jnp.bfloat16))
        lb = (NSTAGE - 1) % 2
        _wslice(w1_any, NSTAGE - 1, stgs[lb], sems1[lb]).wait()
        w1bf[pl.ds((NSTAGE - 1) * STAGE, STAGE), :] = (
            stgs[lb][...].astype(jnp.bfloat16))
        _wslice(w2_any, 0, stgs[0], sems2[0]).start()
        _wslice(w2_any, 1, stgs[1], sems2[1]).start()

    h = lax.dot_general(x_ref[...], w1bf[...], (((1,), (1,)), ((), ())),
                        preferred_element_type=jnp.float32)
    h = jnp.maximum(h + b1_ref[...], 0.0).astype(jnp.bfloat16)

    @pl.when(i == 0)
    def _init_w2():
        for s in range(NSTAGE):
            b = s % 2
            _wslice(w2_any, s, stgs[b], sems2[b]).wait()
            w2bf[pl.ds(s * STAGE, STAGE), :] = (
                stgs[b][...].astype(jnp.bfloat16))
            if s + 2 < NSTAGE:
                _wslice(w2_any, s + 2, stgs[b], sems2[b]).start()

    y = lax.dot_general(h, w2bf[...], (((1,), (0,)), ((), ())),
                        preferred_element_type=jnp.float32)
    o_ref[...] = y + b2_ref[...]


_mlp = pl.pallas_call(
    _mlp_body,
    grid=(N_TOK // BM,),
    in_specs=[
        pl.BlockSpec((BM, IN_F), lambda i: (i, 0)),
        pl.BlockSpec(memory_space=pl.ANY),
        pl.BlockSpec((1, K_SEL), lambda i: (0, 0)),
        pl.BlockSpec(memory_space=pl.ANY),
        pl.BlockSpec((1, OUT_F), lambda i: (0, 0)),
    ],
    out_specs=pl.BlockSpec((BM, OUT_F), lambda i: (i, 0)),
    out_shape=jax.ShapeDtypeStruct((N_TOK, OUT_F), jnp.float32),
    scratch_shapes=[
        pltpu.VMEM((K_SEL, IN_F), jnp.bfloat16),
        pltpu.VMEM((K_SEL, OUT_F), jnp.bfloat16),
        pltpu.VMEM((STAGE, IN_F), jnp.float32),
        pltpu.VMEM((STAGE, IN_F), jnp.float32),
        pltpu.SemaphoreType.DMA,
        pltpu.SemaphoreType.DMA,
        pltpu.SemaphoreType.DMA,
        pltpu.SemaphoreType.DMA,
    ],
    compiler_params=pltpu.CompilerParams(
        dimension_semantics=("arbitrary",),
        vmem_limit_bytes=120 * 1024 * 1024,
    ),
)


def kernel(x, index_vec, fc1_w, fc1_b, fc2_w_t, fc2_b):
    idx = index_vec.astype(jnp.int32)
    w1_sel, b1_sel, w2_sel = _get_sc_gather()(fc1_w, fc1_b, fc2_w_t, idx)
    xbf = _cast_x(x)
    return _mlp(xbf, w1_sel, b1_sel.reshape(1, K_SEL), w2_sel,
                fc2_b.reshape(1, OUT_F))


# speedup vs baseline: 1.8164x; 1.1429x over previous
import functools

import jax
import jax.numpy as jnp
from jax import lax
from jax.experimental import pallas as pl
from jax.experimental.pallas import tpu as pltpu
from jax.experimental.pallas import tpu_sc as plsc

IN_F = 2048
HID = 8192
OUT_F = 2048
N_TOK = 4096
K_SEL = 2048

NC = 2
NS = 16
NW = NC * NS
ROWS_PER_W = K_SEL // NW
CHUNK = 16
NCHUNK = ROWS_PER_W // CHUNK


def _gather_rows_pipelined(tbl_hbm, out_hbm, base, idx_v, bufs, gsems, osems):
    gathers = [None, None]
    outs = [None, None]
    for t in range(NCHUNK):
        b = t % 2
        if outs[b] is not None:
            outs[b].wait()
        idxs = idx_v[pl.ds(t * CHUNK, CHUNK)]
        gathers[b] = pltpu.async_copy(tbl_hbm.at[idxs], bufs[b], gsems[b])
        if t > 0:
            pb = (t - 1) % 2
            gathers[pb].wait()
            outs[pb] = pltpu.async_copy(
                bufs[pb], out_hbm.at[pl.ds(base + (t - 1) * CHUNK, CHUNK)],
                osems[pb])
    lb = (NCHUNK - 1) % 2
    gathers[lb].wait()
    outs[lb] = pltpu.async_copy(
        bufs[lb], out_hbm.at[pl.ds(base + (NCHUNK - 1) * CHUNK, CHUNK)],
        osems[lb])
    outs[0].wait()
    outs[1].wait()


@functools.cache
def _get_sc_gather_w1b1():
    mesh = plsc.VectorSubcoreMesh(core_axis_name="c", subcore_axis_name="s",
                                  num_cores=NC, num_subcores=NS)

    @functools.partial(
        pl.kernel,
        out_type=(
            jax.ShapeDtypeStruct((K_SEL, IN_F), jnp.float32),
            jax.ShapeDtypeStruct((K_SEL,), jnp.float32),
        ),
        mesh=mesh,
        compiler_params=pltpu.CompilerParams(needs_layout_passes=False),
        scratch_types=[
            pltpu.VMEM((ROWS_PER_W,), jnp.int32),
            pltpu.VMEM((CHUNK, IN_F), jnp.float32),
            pltpu.VMEM((CHUNK, IN_F), jnp.float32),
            pltpu.VMEM((HID,), jnp.float32),
            pltpu.VMEM((ROWS_PER_W,), jnp.float32),
            pltpu.SemaphoreType.DMA,
            pltpu.SemaphoreType.DMA,
            pltpu.SemaphoreType.DMA,
            pltpu.SemaphoreType.DMA,
        ],
    )
    def _sc_gather(fc1_w_hbm, fc1_b_hbm, idx_hbm,
                   w1_out, b1_out,
                   idx_v, bufa, bufb, bias_v, bsel_v,
                   gsem_a, gsem_b, osem_a, osem_b):
        wid = lax.axis_index("s") * NC + lax.axis_index("c")
        base = wid * ROWS_PER_W
        pltpu.sync_copy(idx_hbm.at[pl.ds(base, ROWS_PER_W)], idx_v)

        pltpu.sync_copy(fc1_b_hbm, bias_v)
        for c in range(ROWS_PER_W // 16):
            idxs = idx_v[pl.ds(c * 16, 16)]
            bsel_v[pl.ds(c * 16, 16)] = plsc.load_gather(bias_v, [idxs])
        pltpu.sync_copy(bsel_v, b1_out.at[pl.ds(base, ROWS_PER_W)])

        _gather_rows_pipelined(fc1_w_hbm, w1_out, base, idx_v, (bufa, bufb),
                               (gsem_a, gsem_b), (osem_a, osem_b))

    return _sc_gather


@functools.cache
def _get_sc_gather_w2():
    mesh = plsc.VectorSubcoreMesh(core_axis_name="c", subcore_axis_name="s",
                                  num_cores=NC, num_subcores=NS)

    @functools.partial(
        pl.kernel,
        out_type=jax.ShapeDtypeStruct((K_SEL, OUT_F), jnp.float32),
        mesh=mesh,
        compiler_params=pltpu.CompilerParams(needs_layout_passes=False),
        scratch_types=[
            pltpu.VMEM((ROWS_PER_W,), jnp.int32),
            pltpu.VMEM((CHUNK, OUT_F), jnp.float32),
            pltpu.VMEM((CHUNK, OUT_F), jnp.float32),
            pltpu.SemaphoreType.DMA,
            pltpu.SemaphoreType.DMA,
            pltpu.SemaphoreType.DMA,
            pltpu.SemaphoreType.DMA,
        ],
    )
    def _sc_gather(fc2_w_hbm, idx_hbm, w2_out,
                   idx_v, bufa, bufb,
                   gsem_a, gsem_b, osem_a, osem_b):
        wid = lax.axis_index("s") * NC + lax.axis_index("c")
        base = wid * ROWS_PER_W
        pltpu.sync_copy(idx_hbm.at[pl.ds(base, ROWS_PER_W)], idx_v)
        _gather_rows_pipelined(fc2_w_hbm, w2_out, base, idx_v, (bufa, bufb),
                               (gsem_a, gsem_b), (osem_a, osem_b))

    return _sc_gather


BM = 512


def _mlp1_body(x_ref, w1_ref, b1_ref, o_ref, w1bf):
    @pl.when(pl.program_id(0) == 0)
    def _init():
        w1bf[...] = w1_ref[...].astype(jnp.bfloat16)

    xb = x_ref[...].astype(jnp.bfloat16)
    h = lax.dot_general(xb, w1bf[...], (((1,), (1,)), ((), ())),
                        preferred_element_type=jnp.float32)
    o_ref[...] = jnp.maximum(h + b1_ref[...], 0.0).astype(jnp.bfloat16)


_mlp1 = pl.pallas_call(
    _mlp1_body,
    grid=(N_TOK // BM,),
    in_specs=[
        pl.BlockSpec((BM, IN_F), lambda i: (i, 0)),
        pl.BlockSpec((K_SEL, IN_F), lambda i: (0, 0)),
        pl.BlockSpec((1, K_SEL), lambda i: (0, 0)),
    ],
    out_specs=pl.BlockSpec((BM, K_SEL), lambda i: (i, 0)),
    out_shape=jax.ShapeDtypeStruct((N_TOK, K_SEL), jnp.bfloat16),
    scratch_shapes=[pltpu.VMEM((K_SEL, IN_F), jnp.bfloat16)],
    compiler_params=pltpu.CompilerParams(
        dimension_semantics=("arbitrary",),
        vmem_limit_bytes=120 * 1024 * 1024,
    ),
)


def _mlp2_body(h_ref, w2_ref, b2_ref, o_ref, w2bf):
    @pl.when(pl.program_id(0) == 0)
    def _init():
        w2bf[...] = w2_ref[...].astype(jnp.bfloat16)

    y = lax.dot_general(h_ref[...], w2bf[...], (((1,), (0,)), ((), ())),
                        preferred_element_type=jnp.float32)
    o_ref[...] = y + b2_ref[...]


_mlp2 = pl.pallas_call(
    _mlp2_body,
    grid=(N_TOK // BM,),
    in_specs=[
        pl.BlockSpec((BM, K_SEL), lambda i: (i, 0)),
        pl.BlockSpec((K_SEL, OUT_F), lambda i: (0, 0)),
        pl.BlockSpec((1, OUT_F), lambda i: (0, 0)),
    ],
    out_specs=pl.BlockSpec((BM, OUT_F), lambda i: (i, 0)),
    out_shape=jax.ShapeDtypeStruct((N_TOK, OUT_F), jnp.float32),
    scratch_shapes=[pltpu.VMEM((K_SEL, OUT_F), jnp.bfloat16)],
    compiler_params=pltpu.CompilerParams(
        dimension_semantics=("arbitrary",),
        vmem_limit_bytes=120 * 1024 * 1024,
    ),
)


def kernel(x, index_vec, fc1_w, fc1_b, fc2_w_t, fc2_b):
    idx = index_vec.astype(jnp.int32)
    w1_sel, b1_sel = _get_sc_gather_w1b1()(fc1_w, fc1_b, idx)
    w2_sel = _get_sc_gather_w2()(fc2_w_t, idx)
    h = _mlp1(x, w1_sel, b1_sel.reshape(1, K_SEL))
    return _mlp2(h, w2_sel, fc2_b.reshape(1, OUT_F))
